# baseline (speedup 1.0000x reference)
import functools

import jax
import jax.numpy as jnp
from jax.experimental import pallas as pl
from jax.experimental.pallas import tpu as pltpu

N = 10000
D = 128
BM1 = 2000
BM2 = 400


def _support_body(x_ref, w_ref, out_ref):
    out_ref[...] = jnp.dot(x_ref[...], w_ref[...],
                           preferred_element_type=jnp.float32)


def _gcn_body(adj_ref, sup_ref, x_ref, b_ref, out_ref):
    acc = jnp.dot(adj_ref[...], sup_ref[...],
                  preferred_element_type=jnp.float32)
    out_ref[...] = jnp.maximum(acc + x_ref[...] + b_ref[...], 0.0)


@jax.jit
def kernel(input, adj, W, b):
    x = input
    b2 = b.reshape(1, D)

    support = pl.pallas_call(
        _support_body,
        grid=(N // BM1,),
        in_specs=[
            pl.BlockSpec((BM1, D), lambda i: (i, 0)),
            pl.BlockSpec((D, D), lambda i: (0, 0)),
        ],
        out_specs=pl.BlockSpec((BM1, D), lambda i: (i, 0)),
        out_shape=jax.ShapeDtypeStruct((N, D), jnp.float32),
        compiler_params=pltpu.CompilerParams(
            dimension_semantics=("parallel",),
        ),
    )(x, W)

    out = pl.pallas_call(
        _gcn_body,
        grid=(N // BM2,),
        in_specs=[
            pl.BlockSpec((BM2, N), lambda i: (i, 0)),
            pl.BlockSpec((N, D), lambda i: (0, 0)),
            pl.BlockSpec((BM2, D), lambda i: (i, 0)),
            pl.BlockSpec((1, D), lambda i: (0, 0)),
        ],
        out_specs=pl.BlockSpec((BM2, D), lambda i: (i, 0)),
        out_shape=jax.ShapeDtypeStruct((N, D), jnp.float32),
        compiler_params=pltpu.CompilerParams(
            dimension_semantics=("arbitrary",),
        ),
    )(adj, support, x, b2)

    return out


# speedup vs baseline: 1.0498x; 1.0498x over previous
import jax
import jax.numpy as jnp
from jax.experimental import pallas as pl
from jax.experimental.pallas import tpu as pltpu

N = 10000
D = 128
BM = 400


def _gcn_body(adj_ref, xfull_ref, w_ref, b_ref, xblk_ref, out_ref):
    acc = jnp.dot(adj_ref[...], xfull_ref[...],
                  preferred_element_type=jnp.float32)
    y = jnp.dot(acc, w_ref[...], preferred_element_type=jnp.float32)
    out_ref[...] = jnp.maximum(y + xblk_ref[...] + b_ref[...], 0.0)


@jax.jit
def kernel(input, adj, W, b):
    x = input
    b2 = b.reshape(1, D)

    out = pl.pallas_call(
        _gcn_body,
        grid=(N // BM,),
        in_specs=[
            pl.BlockSpec((BM, N), lambda i: (i, 0)),
            pl.BlockSpec((N, D), lambda i: (0, 0)),
            pl.BlockSpec((D, D), lambda i: (0, 0)),
            pl.BlockSpec((1, D), lambda i: (0, 0)),
            pl.BlockSpec((BM, D), lambda i: (i, 0)),
        ],
        out_specs=pl.BlockSpec((BM, D), lambda i: (i, 0)),
        out_shape=jax.ShapeDtypeStruct((N, D), jnp.float32),
        compiler_params=pltpu.CompilerParams(
            dimension_semantics=("arbitrary",),
        ),
    )(adj, x, W, b2, x)

    return out
